# baseline (speedup 1.0000x reference)
import jax
import jax.numpy as jnp
from jax import lax
from jax.experimental import pallas as pl
from jax.experimental.pallas import tpu as pltpu
from jax.experimental.pallas import tpu_sc as plsc

B = 16384
N_NUM = 13
NUM_OUT = 64
N_CAT = 26
VOCAB = 1000
EMB = 32
OUT_D = NUM_OUT + N_CAT * EMB

NC = 2
NS = 16
NW = NC * NS
ROWS_PER_W = B // NW
CHUNK = 128
N_CHUNKS = ROWS_PER_W // CHUNK
LANES = 16
NGROUP = CHUNK // LANES


def _sc_body(num_hbm, cat_hbm, w_hbm, b_hbm, tables_hbm, out_hbm,
             cat_v, idxT_v, dest_v, numin_v, numout_v, w_v, b_v,
             gsem, osem, ssem):
    cid = lax.axis_index("c")
    sid = lax.axis_index("s")
    wid = sid * NC + cid
    row0 = wid * ROWS_PER_W

    pltpu.sync_copy(w_hbm, w_v)
    pltpu.sync_copy(b_hbm, b_v)

    lane = lax.iota(jnp.int32, LANES)

    def chunk_body(ci, carry):
        base = pl.multiple_of(row0 + ci * CHUNK, CHUNK)
        pltpu.sync_copy(cat_hbm.at[pl.ds(base, CHUNK)], cat_v)
        stg = pltpu.async_copy(
            num_hbm.at[pl.ds(base * N_NUM, CHUNK * N_NUM)],
            numin_v.at[pl.ds(0, CHUNK * N_NUM)],
            ssem,
        )
        for g in range(NGROUP):
            rows = lane + (g * LANES)
            for f in range(N_CAT):
                vals = plsc.load_gather(
                    cat_v, [rows, jnp.full((LANES,), f, jnp.int32)]
                )
                idxT_v[f, pl.ds(g * LANES, LANES)] = vals + (f * VOCAB)
        gathers = [
            pltpu.async_copy(tables_hbm.at[idxT_v.at[f]], dest_v.at[f], gsem)
            for f in range(N_CAT)
        ]
        stg.wait()

        def row_body(r, c):
            accs = [b_v[pl.ds(j * LANES, LANES)] for j in range(NUM_OUT // LANES)]
            rowvec = numin_v[pl.ds(r * N_NUM, LANES)]
            for k in range(N_NUM):
                s = rowvec[k]
                for j in range(NUM_OUT // LANES):
                    accs[j] = accs[j] + w_v[k, pl.ds(j * LANES, LANES)] * s
            for j in range(NUM_OUT // LANES):
                numout_v[r, pl.ds(j * LANES, LANES)] = accs[j]
            return c

        lax.fori_loop(0, CHUNK, row_body, 0)
        out_num = pltpu.async_copy(
            numout_v, out_hbm.at[pl.ds(base, CHUNK), pl.ds(0, NUM_OUT)], ssem
        )
        outs = []
        for f in range(N_CAT):
            gathers[f].wait()
            outs.append(
                pltpu.async_copy(
                    dest_v.at[f],
                    out_hbm.at[
                        pl.ds(base, CHUNK), pl.ds(NUM_OUT + f * EMB, EMB)
                    ],
                    osem,
                )
            )
        out_num.wait()
        for o in outs:
            o.wait()
        return carry

    lax.fori_loop(0, N_CHUNKS, chunk_body, 0)


_sc_kernel = pl.kernel(
    _sc_body,
    mesh=plsc.VectorSubcoreMesh(core_axis_name="c", subcore_axis_name="s"),
    compiler_params=pltpu.CompilerParams(
        use_tc_tiling_on_sc=False, needs_layout_passes=False
    ),
    out_type=jax.ShapeDtypeStruct((B, OUT_D), jnp.float32),
    scratch_types=[
        pltpu.VMEM((CHUNK, N_CAT), jnp.int32),
        pltpu.VMEM((N_CAT, CHUNK), jnp.int32),
        pltpu.VMEM((N_CAT, CHUNK, EMB), jnp.float32),
        pltpu.VMEM((CHUNK * N_NUM + LANES,), jnp.float32),
        pltpu.VMEM((CHUNK, NUM_OUT), jnp.float32),
        pltpu.VMEM((N_NUM, NUM_OUT), jnp.float32),
        pltpu.VMEM((NUM_OUT,), jnp.float32),
        pltpu.SemaphoreType.DMA,
        pltpu.SemaphoreType.DMA,
        pltpu.SemaphoreType.DMA,
    ],
)


@jax.jit
def kernel(num_tensor, cat_tensor, W, b, tables):
    tables_flat = tables.reshape(N_CAT * VOCAB, EMB)
    return _sc_kernel(num_tensor.reshape(-1), cat_tensor, W, b, tables_flat)
